# baseline (speedup 1.0000x reference)
import functools

import jax
import jax.numpy as jnp
from jax import lax
from jax.experimental import pallas as pl
from jax.experimental.pallas import tpu as pltpu
from jax.experimental.pallas import tpu_sc as plsc

WEIGHT = 5.0


def _sc_gather(table, idx):
    B = idx.shape[0]
    E = table.shape[1]
    info = plsc.get_sparse_core_info()
    nc, ns = info.num_cores, info.num_subcores
    nw = nc * ns
    b_per_w = B // nw
    mesh = plsc.VectorSubcoreMesh(core_axis_name="c", subcore_axis_name="s")

    @functools.partial(
        pl.kernel,
        mesh=mesh,
        out_type=jax.ShapeDtypeStruct((B, E), jnp.float32),
        scratch_types=[
            pltpu.VMEM((b_per_w,), jnp.int32),
            pltpu.VMEM((b_per_w, E), jnp.float32),
            pltpu.SemaphoreType.DMA,
        ],
    )
    def gather_kernel(table_hbm, idx_hbm, out_hbm, idx_v, rows_v, sem):
        wid = lax.axis_index("s") * nc + lax.axis_index("c")
        base = wid * b_per_w
        pltpu.sync_copy(idx_hbm.at[pl.ds(base, b_per_w)], idx_v)
        pltpu.async_copy(table_hbm.at[idx_v], rows_v, sem).wait()
        pltpu.sync_copy(rows_v, out_hbm.at[pl.ds(base, b_per_w)])

    return gather_kernel(table, idx)


NBUF = 3
VT = 3456


def _compute_tile(x, w):
    xs = ((2.0 * WEIGHT) * x).astype(jnp.bfloat16)
    wb = w.astype(jnp.bfloat16)
    t2 = lax.dot_general(xs, wb, (((1,), (1,)), ((), ())),
                         preferred_element_type=jnp.float32)
    negw = jnp.full((1, w.shape[1]), -WEIGHT, jnp.bfloat16)
    t1row = lax.dot_general(negw, wb * wb, (((1,), (1,)), ((), ())),
                            preferred_element_type=jnp.float32)
    t3 = jnp.sum(x * x, axis=1)
    return t2 + t1row + (-WEIGHT) * t3[:, None]


def _make_body(NT, TAIL, E):
    TAILA = TAIL - 1

    def body(x_ref, w_any, wflat_any, o_ref, wbuf, wrow, rsem):
        i = pl.program_id(0)
        slot_r = lax.rem(i, NBUF)

        def rd_full(j, slot):
            return pltpu.make_async_copy(
                w_any.at[pl.ds(j * VT, VT)], wbuf.at[slot], rsem.at[slot])

        def rd_tail_a(j, slot):
            return pltpu.make_async_copy(
                w_any.at[pl.ds(j * VT, TAILA)],
                wbuf.at[slot, pl.ds(0, TAILA)], rsem.at[slot])

        def rd_tail_b(slot):
            return pltpu.make_async_copy(
                wflat_any.at[pl.ds(((NT - 1) * VT + TAILA) * E, E)],
                wrow, rsem.at[slot])

        @pl.when(i == 0)
        def _prologue():
            rd_full(0, 0).start()
            rd_full(1, 1).start()

        @pl.when(i + 2 < NT - 1)
        def _():
            rd_full(i + 2, lax.rem(i + 2, NBUF)).start()

        @pl.when(i + 2 == NT - 1)
        def _():
            s = lax.rem(i + 2, NBUF)
            rd_tail_a(i + 2, s).start()
            rd_tail_b(s).start()

        @pl.when(i < NT - 1)
        def _():
            rd_full(i, slot_r).wait()

        @pl.when(i == NT - 1)
        def _():
            rd_tail_a(i, slot_r).wait()
            rd_tail_b(slot_r).wait()
            wbuf[slot_r, TAILA:TAILA + 1, :] = wrow[...].reshape(1, E)

        x = x_ref[...]
        w = wbuf[slot_r]
        o_ref[...] = _compute_tile(x, w)

    return body


def _tc_bias(cur, embed_weight, interpret=False):
    B, E = cur.shape
    V = embed_weight.shape[0]
    NT = pl.cdiv(V, VT)
    TAIL = V - (NT - 1) * VT
    wflat = embed_weight.reshape(V * E)
    out = pl.pallas_call(
        _make_body(NT, TAIL, E),
        grid=(NT,),
        in_specs=[
            pl.BlockSpec((B, E), lambda i: (0, 0)),
            pl.BlockSpec(memory_space=pl.ANY),
            pl.BlockSpec(memory_space=pl.ANY),
        ],
        out_specs=pl.BlockSpec((B, VT), lambda i: (0, i)),
        out_shape=jax.ShapeDtypeStruct((B, V), jnp.float32),
        scratch_shapes=[
            pltpu.VMEM((NBUF, VT, E), jnp.float32),
            pltpu.VMEM((E,), jnp.float32),
            pltpu.SemaphoreType.DMA((NBUF,)),
        ],
        compiler_params=pltpu.CompilerParams(
            vmem_limit_bytes=64 * 1024 * 1024),
        interpret=interpret,
    )(cur, embed_weight, wflat)
    return out


def kernel(sampled_ids, embed_weight):
    Bt, S = sampled_ids.shape
    V, E = embed_weight.shape
    B = Bt * S
    idx = sampled_ids.reshape(B).astype(jnp.int32)

    cur = _sc_gather(embed_weight, idx)
    out = _tc_bias(cur, embed_weight)
    return out.reshape(Bt, S, V)


# speedup vs baseline: 2.1923x; 2.1923x over previous
import functools

import jax
import jax.numpy as jnp
from jax import lax
from jax.experimental import pallas as pl
from jax.experimental.pallas import tpu as pltpu
from jax.experimental.pallas import tpu_sc as plsc

WEIGHT = 5.0


def _sc_gather(table, idx):
    B = idx.shape[0]
    E = table.shape[1]
    info = plsc.get_sparse_core_info()
    nc, ns = info.num_cores, info.num_subcores
    nw = nc * ns
    b_per_w = B // nw
    mesh = plsc.VectorSubcoreMesh(core_axis_name="c", subcore_axis_name="s")

    @functools.partial(
        pl.kernel,
        mesh=mesh,
        out_type=jax.ShapeDtypeStruct((B, E), jnp.float32),
        scratch_types=[
            pltpu.VMEM((b_per_w,), jnp.int32),
            pltpu.VMEM((b_per_w, E), jnp.float32),
            pltpu.SemaphoreType.DMA,
        ],
    )
    def gather_kernel(table_hbm, idx_hbm, out_hbm, idx_v, rows_v, sem):
        wid = lax.axis_index("s") * nc + lax.axis_index("c")
        base = wid * b_per_w
        pltpu.sync_copy(idx_hbm.at[pl.ds(base, b_per_w)], idx_v)
        pltpu.async_copy(table_hbm.at[idx_v], rows_v, sem).wait()
        pltpu.sync_copy(rows_v, out_hbm.at[pl.ds(base, b_per_w)])

    return gather_kernel(table, idx)


def _bias_body(x_ref, w_ref, o_ref):
    x = x_ref[...]
    w = w_ref[...]
    xs = ((2.0 * WEIGHT) * x).astype(jnp.bfloat16)
    wb = w.astype(jnp.bfloat16)
    t2 = lax.dot_general(xs, wb, (((1,), (1,)), ((), ())),
                         preferred_element_type=jnp.float32)
    negw = jnp.full((1, w.shape[1]), -WEIGHT, jnp.bfloat16)
    t1row = lax.dot_general(negw, wb * wb, (((1,), (1,)), ((), ())),
                            preferred_element_type=jnp.float32)
    t3 = jnp.sum(x * x, axis=1)
    o_ref[...] = t2 + t1row + (-WEIGHT) * t3[:, None]


def kernel(sampled_ids, embed_weight):
    Bt, S = sampled_ids.shape
    V, E = embed_weight.shape
    B = Bt * S
    idx = sampled_ids.reshape(B).astype(jnp.int32)

    cur = _sc_gather(embed_weight, idx)

    VT = 4736
    out = pl.pallas_call(
        _bias_body,
        grid=(pl.cdiv(V, VT),),
        compiler_params=pltpu.CompilerParams(
            vmem_limit_bytes=64 * 1024 * 1024),
        in_specs=[
            pl.BlockSpec((B, E), lambda i: (0, 0)),
            pl.BlockSpec((VT, E), lambda i: (i, 0)),
        ],
        out_specs=pl.BlockSpec((B, VT), lambda i: (0, i)),
        out_shape=jax.ShapeDtypeStruct((B, V), jnp.float32),
    )(cur, embed_weight)

    return out.reshape(Bt, S, V)
